# baseline (speedup 1.0000x reference)
import functools

import jax
import jax.numpy as jnp
from jax import lax
from jax.experimental import pallas as pl
from jax.experimental.pallas import tpu as pltpu
from jax.experimental.pallas import tpu_sc as plsc

N = 10000
E = 160000
D = 128
S = 4
NRB = 8
H = 64
NE = 4
DSC = 128
AVG_INV = 1.0 / 16.0

NC, NS = 2, 16
NW = NC * NS
CH = 128
NP = 10240
RZ = NP // NS
NCHUNK = 1
EC = E // NCHUNK


@functools.cache
def _sc_mesh():
  return plsc.VectorSubcoreMesh(
      core_axis_name="c", subcore_axis_name="s", num_cores=NC, num_subcores=NS)


def _final_body(p_ref, natot_ref, wsc_ref, mi_ref, scs_ref):
  m = p_ref[0] + p_ref[1]
  mi_ref[...] = m
  acc = jnp.zeros((m.shape[0], DSC), jnp.float32)
  for q in range(NE):
    acc += jnp.dot(m, wsc_ref[q], preferred_element_type=jnp.float32) \
        * natot_ref[:, q:q + 1]
  scs_ref[...] = acc


def _final_stage(partials, natot, W_sc):
  BN = 2000
  grid = (N // BN,)
  return pl.pallas_call(
      _final_body,
      grid=grid,
      in_specs=[
          pl.BlockSpec((NC, BN, D), lambda i: (0, i, 0)),
          pl.BlockSpec((BN, NE), lambda i: (i, 0)),
          pl.BlockSpec((NE, D, DSC), lambda i: (0, 0, 0)),
      ],
      out_specs=[
          pl.BlockSpec((BN, D), lambda i: (i, 0)),
          pl.BlockSpec((BN, DSC), lambda i: (i, 0)),
      ],
      out_shape=[
          jax.ShapeDtypeStruct((N, D), jnp.float32),
          jax.ShapeDtypeStruct((N, DSC), jnp.float32),
      ],
  )(partials, natot, W_sc)


def _edge_body(ef_ref, attr_ref, cut_ref, xe_ref, wup_ref, wr0_ref, wr1_ref,
               wr2_ref, wlin_ref, v_ref):
  h = jax.nn.silu(jnp.dot(ef_ref[...], wr0_ref[...],
                          preferred_element_type=jnp.float32))
  h = jax.nn.silu(jnp.dot(h, wr1_ref[...], preferred_element_type=jnp.float32))
  cw = jnp.dot(h.astype(jnp.bfloat16), wr2_ref[...].astype(jnp.bfloat16),
               preferred_element_type=jnp.float32)
  xe = jnp.dot(xe_ref[...].astype(jnp.bfloat16),
               wup_ref[...].astype(jnp.bfloat16),
               preferred_element_type=jnp.float32)
  ac = attr_ref[...] * cut_ref[...]
  ts = [(xe * (cw[:, s * D:(s + 1) * D] * ac[:, s:s + 1])).astype(jnp.bfloat16)
        for s in range(S)]
  t = jnp.concatenate(ts, axis=1)
  wl = wlin_ref[...].reshape(S * D, D).astype(jnp.bfloat16)
  acc = jnp.dot(t, wl, preferred_element_type=jnp.float32)
  v_ref[...] = acc * AVG_INV


def _edge_stage(edge_feats, edge_attrs, cutoff, xe, W_up, Wr0, Wr1, Wr2p,
                W_linp):
  BE = 2000
  ec = edge_feats.shape[0]
  grid = (ec // BE,)
  return pl.pallas_call(
      _edge_body,
      grid=grid,
      in_specs=[
          pl.BlockSpec((BE, NRB), lambda i: (i, 0)),
          pl.BlockSpec((BE, S), lambda i: (i, 0)),
          pl.BlockSpec((BE, 1), lambda i: (i, 0)),
          pl.BlockSpec((BE, D), lambda i: (i, 0)),
          pl.BlockSpec((D, D), lambda i: (0, 0)),
          pl.BlockSpec((NRB, H), lambda i: (0, 0)),
          pl.BlockSpec((H, H), lambda i: (0, 0)),
          pl.BlockSpec((H, S * D), lambda i: (0, 0)),
          pl.BlockSpec((S, D, D), lambda i: (0, 0, 0)),
      ],
      out_specs=pl.BlockSpec((BE, D), lambda i: (i, 0)),
      out_shape=jax.ShapeDtypeStruct((ec, D), jnp.float32),
  )(edge_feats, edge_attrs, cutoff, xe, W_up, Wr0, Wr1, Wr2p, W_linp)


@functools.cache
def _build_gather(ec):
  nch = ec // CH
  nfull = nch // NW
  remw = 16
  remr = (nch - nfull * NW) * CH // remw
  rbase = nfull * NW * CH
  assert nfull % 2 == 1 and remr % 16 == 0 and 0 < remr <= CH

  def body(x_hbm, src_hbm, xe_hbm, idx0, idx1, rows0, rows1, idx_t, rows_t,
           sem0, sem1, sem_t):
    wid = lax.axis_index("s") * NC + lax.axis_index("c")
    idx = (idx0, idx1)
    rows = (rows0, rows1)
    sems = (sem0, sem1)

    def fire(p, c):
      off = (c * NW + wid) * CH
      pltpu.sync_copy(src_hbm.at[pl.ds(off, CH)], idx[p])
      pltpu.async_copy(x_hbm.at[idx[p]], rows[p], sems[p])

    def drain(p, c):
      off = (c * NW + wid) * CH
      pltpu.make_async_copy(x_hbm.at[idx[p]], rows[p], sems[p]).wait()
      pltpu.sync_copy(rows[p], xe_hbm.at[pl.ds(off, CH)])

    for p in range(2):
      fire(p, p)

    def pair(t, carry):
      c = 2 * t
      for p in range(2):
        drain(p, c + p)
        fire(p, c + 2 + p)
      return carry

    lax.fori_loop(0, (nfull - 3) // 2, pair, 0)
    drain(0, nfull - 3)
    fire(0, nfull - 1)
    drain(1, nfull - 2)

    @pl.when(wid < remw)
    def _tail_fire():
      off = rbase + wid * remr
      pltpu.sync_copy(src_hbm.at[pl.ds(off, remr)], idx_t)
      pltpu.async_copy(x_hbm.at[idx_t], rows_t, sem_t)

    drain(0, nfull - 1)

    @pl.when(wid < remw)
    def _tail_drain():
      off = rbase + wid * remr
      pltpu.make_async_copy(x_hbm.at[idx_t], rows_t, sem_t).wait()
      pltpu.sync_copy(rows_t, xe_hbm.at[pl.ds(off, remr)])

  return functools.partial(
      pl.kernel,
      out_type=jax.ShapeDtypeStruct((ec, D), jnp.float32),
      mesh=_sc_mesh(),
      scratch_types=[
          pltpu.VMEM((CH,), jnp.int32),
          pltpu.VMEM((CH,), jnp.int32),
          pltpu.VMEM((CH, D), jnp.float32),
          pltpu.VMEM((CH, D), jnp.float32),
          pltpu.VMEM((remr,), jnp.int32),
          pltpu.VMEM((remr, D), jnp.float32),
          pltpu.SemaphoreType.DMA,
          pltpu.SemaphoreType.DMA,
          pltpu.SemaphoreType.DMA,
      ],
  )(body)


@functools.cache
def _build_scatter(ec, nchunk):
  epw = ec // NW
  nfull = epw // CH
  tail = epw - nfull * CH
  assert epw % 8 == 0 and tail % 8 == 0 and 0 < tail <= CH and nfull % 2 == 1

  def body(*refs):
    v_hbms = refs[:nchunk]
    (dst_hbm, zeros_hbm, out_hbm, idx0, idx1, rows0, rows1, idx_t, rows_t,
     acc_sh, sem0, sem1, sem_t) = refs[nchunk:]
    cid = lax.axis_index("c")
    sid = lax.axis_index("s")
    pltpu.sync_copy(zeros_hbm.at[pl.ds(sid * RZ, RZ)],
                    acc_sh.at[pl.ds(sid * RZ, RZ)])
    plsc.subcore_barrier()

    wid = sid * NC + cid
    idx = (idx0, idx1)
    rows = (rows0, rows1)
    sems = (sem0, sem1)

    for k in range(nchunk):
      v_hbm = v_hbms[k]
      base = wid * epw
      gbase = k * ec + base

      def fire(p, c):
        pltpu.sync_copy(dst_hbm.at[pl.ds(gbase + c * CH, CH)], idx[p])
        pltpu.async_copy(v_hbm.at[pl.ds(base + c * CH, CH)], rows[p], sems[p])

      def drain(p, c):
        pltpu.make_async_copy(v_hbm.at[pl.ds(base + c * CH, CH)], rows[p],
                              sems[p]).wait()
        pltpu.sync_copy(rows[p], acc_sh.at[idx[p]], add=True)

      for p in range(2):
        fire(p, p)

      def pair(t, carry):
        c = 2 * t
        for p in range(2):
          drain(p, c + p)
          fire(p, c + 2 + p)
        return carry

      lax.fori_loop(0, (nfull - 3) // 2, pair, 0)
      drain(0, nfull - 3)
      fire(0, nfull - 1)
      drain(1, nfull - 2)
      pltpu.sync_copy(dst_hbm.at[pl.ds(gbase + nfull * CH, tail)], idx_t)
      pltpu.async_copy(v_hbm.at[pl.ds(base + nfull * CH, tail)], rows_t, sem_t)
      drain(0, nfull - 1)
      pltpu.make_async_copy(v_hbm.at[pl.ds(base + nfull * CH, tail)], rows_t,
                            sem_t).wait()
      pltpu.sync_copy(rows_t, acc_sh.at[idx_t], add=True)

    plsc.subcore_barrier()
    pltpu.sync_copy(acc_sh.at[pl.ds(sid * RZ, RZ)],
                    out_hbm.at[cid, pl.ds(sid * RZ, RZ)])

  return functools.partial(
      pl.kernel,
      out_type=jax.ShapeDtypeStruct((NC, NP, D), jnp.float32),
      mesh=_sc_mesh(),
      scratch_types=[
          pltpu.VMEM((CH,), jnp.int32),
          pltpu.VMEM((CH,), jnp.int32),
          pltpu.VMEM((CH, D), jnp.float32),
          pltpu.VMEM((CH, D), jnp.float32),
          pltpu.VMEM((tail,), jnp.int32),
          pltpu.VMEM((tail, D), jnp.float32),
          pltpu.VMEM_SHARED((NP, D), jnp.float32),
          pltpu.SemaphoreType.DMA,
          pltpu.SemaphoreType.DMA,
          pltpu.SemaphoreType.DMA,
      ],
  )(body)


def kernel(node_feats, node_attrs_total, node_attrs_slice, edge_feats,
           edge_attrs, edge_index, cutoff, W_up, Wr0, Wr1, Wr2, W_lin, W_sc):
  del node_attrs_slice
  src = edge_index[0]
  dst = edge_index[1]
  Wr2p = Wr2.reshape(H, D, S).transpose(0, 2, 1).reshape(H, S * D)
  W_linp = W_lin.reshape(D, S, D).transpose(1, 0, 2)

  gather = _build_gather(EC)
  vs = []
  for k in range(NCHUNK):
    lo, hi = k * EC, (k + 1) * EC
    xe = gather(node_feats, src[lo:hi])
    vs.append(_edge_stage(edge_feats[lo:hi], edge_attrs[lo:hi], cutoff[lo:hi],
                          xe, W_up, Wr0, Wr1, Wr2p, W_linp))
  zeros = jnp.zeros((NP, D), jnp.float32)
  partial = _build_scatter(EC, NCHUNK)(*vs, dst, zeros)
  m_i, scs = _final_stage(partial, node_attrs_total, W_sc)
  return (m_i, scs)


# speedup vs baseline: 1.0944x; 1.0944x over previous
import functools

import jax
import jax.numpy as jnp
from jax import lax
from jax.experimental import pallas as pl
from jax.experimental.pallas import tpu as pltpu
from jax.experimental.pallas import tpu_sc as plsc

N = 10000
E = 160000
D = 128
S = 4
NRB = 8
H = 64
NE = 4
DSC = 128
AVG_INV = 1.0 / 16.0

NC, NS = 2, 16
NW = NC * NS
CH = 128
NP = 10240
RZ = NP // NS
NCHUNK = 1
EC = E // NCHUNK


@functools.cache
def _sc_mesh():
  return plsc.VectorSubcoreMesh(
      core_axis_name="c", subcore_axis_name="s", num_cores=NC, num_subcores=NS)


def _mm_body(a_ref, b_ref, o_ref):
  o_ref[...] = jnp.dot(a_ref[...], b_ref[...], preferred_element_type=jnp.float32)


def _linear_up(node_feats, W_up):
  return pl.pallas_call(
      _mm_body,
      out_shape=jax.ShapeDtypeStruct((N, D), jnp.float32),
  )(node_feats, W_up)


def _final_body(p_ref, natot_ref, wsc_ref, mi_ref, scs_ref):
  m = p_ref[0] + p_ref[1]
  mi_ref[...] = m
  acc = jnp.zeros((m.shape[0], DSC), jnp.float32)
  for q in range(NE):
    acc += jnp.dot(m, wsc_ref[q], preferred_element_type=jnp.float32) \
        * natot_ref[:, q:q + 1]
  scs_ref[...] = acc


def _final_stage(partials, natot, W_sc):
  BN = 2000
  grid = (N // BN,)
  return pl.pallas_call(
      _final_body,
      grid=grid,
      in_specs=[
          pl.BlockSpec((NC, BN, D), lambda i: (0, i, 0)),
          pl.BlockSpec((BN, NE), lambda i: (i, 0)),
          pl.BlockSpec((NE, D, DSC), lambda i: (0, 0, 0)),
      ],
      out_specs=[
          pl.BlockSpec((BN, D), lambda i: (i, 0)),
          pl.BlockSpec((BN, DSC), lambda i: (i, 0)),
      ],
      out_shape=[
          jax.ShapeDtypeStruct((N, D), jnp.float32),
          jax.ShapeDtypeStruct((N, DSC), jnp.float32),
      ],
  )(partials, natot, W_sc)


def _edge_body(ef_ref, attr_ref, cut_ref, xe_ref, wr0_ref, wr1_ref,
               wr2_ref, wlin_ref, v_ref):
  h = jax.nn.silu(jnp.dot(ef_ref[...], wr0_ref[...],
                          preferred_element_type=jnp.float32))
  h = jax.nn.silu(jnp.dot(h, wr1_ref[...], preferred_element_type=jnp.float32))
  cw = jnp.dot(h.astype(jnp.bfloat16), wr2_ref[...].astype(jnp.bfloat16),
               preferred_element_type=jnp.float32)
  xe = xe_ref[...]
  ac = attr_ref[...] * cut_ref[...]
  ts = [(xe * (cw[:, s * D:(s + 1) * D] * ac[:, s:s + 1])).astype(jnp.bfloat16)
        for s in range(S)]
  t = jnp.concatenate(ts, axis=1)
  wl = wlin_ref[...].reshape(S * D, D).astype(jnp.bfloat16)
  acc = jnp.dot(t, wl, preferred_element_type=jnp.float32)
  v_ref[...] = acc * AVG_INV


def _edge_stage(edge_feats, edge_attrs, cutoff, xe, Wr0, Wr1, Wr2p, W_linp):
  BE = 4000
  ec = edge_feats.shape[0]
  grid = (ec // BE,)
  return pl.pallas_call(
      _edge_body,
      grid=grid,
      in_specs=[
          pl.BlockSpec((BE, NRB), lambda i: (i, 0)),
          pl.BlockSpec((BE, S), lambda i: (i, 0)),
          pl.BlockSpec((BE, 1), lambda i: (i, 0)),
          pl.BlockSpec((BE, D), lambda i: (i, 0)),
          pl.BlockSpec((NRB, H), lambda i: (0, 0)),
          pl.BlockSpec((H, H), lambda i: (0, 0)),
          pl.BlockSpec((H, S * D), lambda i: (0, 0)),
          pl.BlockSpec((S, D, D), lambda i: (0, 0, 0)),
      ],
      out_specs=pl.BlockSpec((BE, D), lambda i: (i, 0)),
      out_shape=jax.ShapeDtypeStruct((ec, D), jnp.float32),
  )(edge_feats, edge_attrs, cutoff, xe, Wr0, Wr1, Wr2p, W_linp)


@functools.cache
def _build_gather(ec):
  nch = ec // CH
  nfull = nch // NW
  remw = 16
  remr = (nch - nfull * NW) * CH // remw
  rbase = nfull * NW * CH
  assert nfull % 2 == 1 and remr % 16 == 0 and 0 < remr <= CH

  def body(x_hbm, src_hbm, xe_hbm, idx0, idx1, rows0, rows1, idx_t, rows_t,
           sem0, sem1, sem_t):
    wid = lax.axis_index("s") * NC + lax.axis_index("c")
    idx = (idx0, idx1)
    rows = (rows0, rows1)
    sems = (sem0, sem1)

    def fire(p, c):
      off = (c * NW + wid) * CH
      pltpu.sync_copy(src_hbm.at[pl.ds(off, CH)], idx[p])
      pltpu.async_copy(x_hbm.at[idx[p]], rows[p], sems[p])

    def drain(p, c):
      off = (c * NW + wid) * CH
      pltpu.make_async_copy(x_hbm.at[idx[p]], rows[p], sems[p]).wait()
      pltpu.sync_copy(rows[p], xe_hbm.at[pl.ds(off, CH)])

    for p in range(2):
      fire(p, p)

    def pair(t, carry):
      c = 2 * t
      for p in range(2):
        drain(p, c + p)
        fire(p, c + 2 + p)
      return carry

    lax.fori_loop(0, (nfull - 3) // 2, pair, 0)
    drain(0, nfull - 3)
    fire(0, nfull - 1)
    drain(1, nfull - 2)

    @pl.when(wid < remw)
    def _tail_fire():
      off = rbase + wid * remr
      pltpu.sync_copy(src_hbm.at[pl.ds(off, remr)], idx_t)
      pltpu.async_copy(x_hbm.at[idx_t], rows_t, sem_t)

    drain(0, nfull - 1)

    @pl.when(wid < remw)
    def _tail_drain():
      off = rbase + wid * remr
      pltpu.make_async_copy(x_hbm.at[idx_t], rows_t, sem_t).wait()
      pltpu.sync_copy(rows_t, xe_hbm.at[pl.ds(off, remr)])

  return functools.partial(
      pl.kernel,
      out_type=jax.ShapeDtypeStruct((ec, D), jnp.float32),
      mesh=_sc_mesh(),
      scratch_types=[
          pltpu.VMEM((CH,), jnp.int32),
          pltpu.VMEM((CH,), jnp.int32),
          pltpu.VMEM((CH, D), jnp.float32),
          pltpu.VMEM((CH, D), jnp.float32),
          pltpu.VMEM((remr,), jnp.int32),
          pltpu.VMEM((remr, D), jnp.float32),
          pltpu.SemaphoreType.DMA,
          pltpu.SemaphoreType.DMA,
          pltpu.SemaphoreType.DMA,
      ],
  )(body)


@functools.cache
def _build_scatter(ec, nchunk):
  epw = ec // NW
  nfull = epw // CH
  tail = epw - nfull * CH
  assert epw % 8 == 0 and tail % 8 == 0 and 0 < tail <= CH and nfull % 2 == 1

  def body(*refs):
    v_hbms = refs[:nchunk]
    (dst_hbm, zeros_hbm, out_hbm, idx0, idx1, rows0, rows1, idx_t, rows_t,
     acc_sh, sem0, sem1, sem_t) = refs[nchunk:]
    cid = lax.axis_index("c")
    sid = lax.axis_index("s")
    pltpu.sync_copy(zeros_hbm.at[pl.ds(sid * RZ, RZ)],
                    acc_sh.at[pl.ds(sid * RZ, RZ)])
    plsc.subcore_barrier()

    wid = sid * NC + cid
    idx = (idx0, idx1)
    rows = (rows0, rows1)
    sems = (sem0, sem1)

    for k in range(nchunk):
      v_hbm = v_hbms[k]
      base = wid * epw
      gbase = k * ec + base

      def fire(p, c):
        pltpu.sync_copy(dst_hbm.at[pl.ds(gbase + c * CH, CH)], idx[p])
        pltpu.async_copy(v_hbm.at[pl.ds(base + c * CH, CH)], rows[p], sems[p])

      def drain(p, c):
        pltpu.make_async_copy(v_hbm.at[pl.ds(base + c * CH, CH)], rows[p],
                              sems[p]).wait()
        pltpu.sync_copy(rows[p], acc_sh.at[idx[p]], add=True)

      for p in range(2):
        fire(p, p)

      def pair(t, carry):
        c = 2 * t
        for p in range(2):
          drain(p, c + p)
          fire(p, c + 2 + p)
        return carry

      lax.fori_loop(0, (nfull - 3) // 2, pair, 0)
      drain(0, nfull - 3)
      fire(0, nfull - 1)
      drain(1, nfull - 2)
      pltpu.sync_copy(dst_hbm.at[pl.ds(gbase + nfull * CH, tail)], idx_t)
      pltpu.async_copy(v_hbm.at[pl.ds(base + nfull * CH, tail)], rows_t, sem_t)
      drain(0, nfull - 1)
      pltpu.make_async_copy(v_hbm.at[pl.ds(base + nfull * CH, tail)], rows_t,
                            sem_t).wait()
      pltpu.sync_copy(rows_t, acc_sh.at[idx_t], add=True)

    plsc.subcore_barrier()
    pltpu.sync_copy(acc_sh.at[pl.ds(sid * RZ, RZ)],
                    out_hbm.at[cid, pl.ds(sid * RZ, RZ)])

  return functools.partial(
      pl.kernel,
      out_type=jax.ShapeDtypeStruct((NC, NP, D), jnp.float32),
      mesh=_sc_mesh(),
      scratch_types=[
          pltpu.VMEM((CH,), jnp.int32),
          pltpu.VMEM((CH,), jnp.int32),
          pltpu.VMEM((CH, D), jnp.float32),
          pltpu.VMEM((CH, D), jnp.float32),
          pltpu.VMEM((tail,), jnp.int32),
          pltpu.VMEM((tail, D), jnp.float32),
          pltpu.VMEM_SHARED((NP, D), jnp.float32),
          pltpu.SemaphoreType.DMA,
          pltpu.SemaphoreType.DMA,
          pltpu.SemaphoreType.DMA,
      ],
  )(body)


def kernel(node_feats, node_attrs_total, node_attrs_slice, edge_feats,
           edge_attrs, edge_index, cutoff, W_up, Wr0, Wr1, Wr2, W_lin, W_sc):
  del node_attrs_slice
  src = edge_index[0]
  dst = edge_index[1]
  Wr2p = Wr2.reshape(H, D, S).transpose(0, 2, 1).reshape(H, S * D)
  W_linp = W_lin.reshape(D, S, D).transpose(1, 0, 2)

  x = _linear_up(node_feats, W_up)
  gather = _build_gather(EC)
  vs = []
  for k in range(NCHUNK):
    lo, hi = k * EC, (k + 1) * EC
    xe = gather(x, src[lo:hi])
    vs.append(_edge_stage(edge_feats[lo:hi], edge_attrs[lo:hi], cutoff[lo:hi],
                          xe, Wr0, Wr1, Wr2p, W_linp))
  zeros = jnp.zeros((NP, D), jnp.float32)
  partial = _build_scatter(EC, NCHUNK)(*vs, dst, zeros)
  m_i, scs = _final_stage(partial, node_attrs_total, W_sc)
  return (m_i, scs)


# speedup vs baseline: 1.1040x; 1.0087x over previous
import functools

import jax
import jax.numpy as jnp
from jax import lax
from jax.experimental import pallas as pl
from jax.experimental.pallas import tpu as pltpu
from jax.experimental.pallas import tpu_sc as plsc

N = 10000
E = 160000
D = 128
S = 4
NRB = 8
H = 64
NE = 4
DSC = 128
AVG_INV = 1.0 / 16.0

NC, NS = 2, 16
NW = NC * NS
CH = 128
NP = 10240
RZ = NP // NS
NCHUNK = 1
EC = E // NCHUNK


@functools.cache
def _sc_mesh():
  return plsc.VectorSubcoreMesh(
      core_axis_name="c", subcore_axis_name="s", num_cores=NC, num_subcores=NS)


def _mm_body(a_ref, b_ref, o_ref):
  o_ref[...] = jnp.dot(a_ref[...], b_ref[...], preferred_element_type=jnp.float32)


def _linear_up(node_feats, W_up):
  return pl.pallas_call(
      _mm_body,
      out_shape=jax.ShapeDtypeStruct((N, D), jnp.float32),
  )(node_feats, W_up)


def _final_body(p_ref, natot_ref, wsc_ref, mi_ref, scs_ref):
  m = p_ref[0] + p_ref[1]
  mi_ref[...] = m
  acc = jnp.zeros((m.shape[0], DSC), jnp.float32)
  for q in range(NE):
    acc += jnp.dot(m, wsc_ref[q], preferred_element_type=jnp.float32) \
        * natot_ref[:, q:q + 1]
  scs_ref[...] = acc


def _final_stage(partials, natot, W_sc):
  BN = 2000
  grid = (N // BN,)
  return pl.pallas_call(
      _final_body,
      grid=grid,
      in_specs=[
          pl.BlockSpec((NC, BN, D), lambda i: (0, i, 0)),
          pl.BlockSpec((BN, NE), lambda i: (i, 0)),
          pl.BlockSpec((NE, D, DSC), lambda i: (0, 0, 0)),
      ],
      out_specs=[
          pl.BlockSpec((BN, D), lambda i: (i, 0)),
          pl.BlockSpec((BN, DSC), lambda i: (i, 0)),
      ],
      out_shape=[
          jax.ShapeDtypeStruct((N, D), jnp.float32),
          jax.ShapeDtypeStruct((N, DSC), jnp.float32),
      ],
  )(partials, natot, W_sc)


def _edge_body(ef_ref, attr_ref, cut_ref, xe_ref, wr0_ref, wr1_ref,
               wr2_ref, wlin_ref, v_ref):
  h = jax.nn.silu(jnp.dot(ef_ref[...], wr0_ref[...],
                          preferred_element_type=jnp.float32))
  h = jax.nn.silu(jnp.dot(h, wr1_ref[...], preferred_element_type=jnp.float32))
  cw = jnp.dot(h.astype(jnp.bfloat16), wr2_ref[...].astype(jnp.bfloat16),
               preferred_element_type=jnp.float32)
  xe = xe_ref[...]
  ac = attr_ref[...] * cut_ref[...]
  ts = [(xe * (cw[:, s * D:(s + 1) * D] * ac[:, s:s + 1])).astype(jnp.bfloat16)
        for s in range(S)]
  t = jnp.concatenate(ts, axis=1)
  wl = wlin_ref[...].reshape(S * D, D).astype(jnp.bfloat16)
  acc = jnp.dot(t, wl, preferred_element_type=jnp.float32)
  v_ref[...] = acc * AVG_INV


def _edge_stage(edge_feats, edge_attrs, cutoff, xe, Wr0, Wr1, Wr2p, W_linp):
  BE = 8000
  ec = edge_feats.shape[0]
  grid = (ec // BE,)
  return pl.pallas_call(
      _edge_body,
      grid=grid,
      in_specs=[
          pl.BlockSpec((BE, NRB), lambda i: (i, 0)),
          pl.BlockSpec((BE, S), lambda i: (i, 0)),
          pl.BlockSpec((BE, 1), lambda i: (i, 0)),
          pl.BlockSpec((BE, D), lambda i: (i, 0)),
          pl.BlockSpec((NRB, H), lambda i: (0, 0)),
          pl.BlockSpec((H, H), lambda i: (0, 0)),
          pl.BlockSpec((H, S * D), lambda i: (0, 0)),
          pl.BlockSpec((S, D, D), lambda i: (0, 0, 0)),
      ],
      out_specs=pl.BlockSpec((BE, D), lambda i: (i, 0)),
      out_shape=jax.ShapeDtypeStruct((ec, D), jnp.float32),
  )(edge_feats, edge_attrs, cutoff, xe, Wr0, Wr1, Wr2p, W_linp)


@functools.cache
def _build_gather(ec):
  nch = ec // CH
  nfull = nch // NW
  remw = 16
  remr = (nch - nfull * NW) * CH // remw
  rbase = nfull * NW * CH
  assert nfull % 2 == 1 and remr % 16 == 0 and 0 < remr <= CH

  def body(x_hbm, src_hbm, xe_hbm, idx0, idx1, rows0, rows1, idx_t, rows_t,
           sem0, sem1, sem_t):
    wid = lax.axis_index("s") * NC + lax.axis_index("c")
    idx = (idx0, idx1)
    rows = (rows0, rows1)
    sems = (sem0, sem1)

    def fire(p, c):
      off = (c * NW + wid) * CH
      pltpu.sync_copy(src_hbm.at[pl.ds(off, CH)], idx[p])
      pltpu.async_copy(x_hbm.at[idx[p]], rows[p], sems[p])

    def drain(p, c):
      off = (c * NW + wid) * CH
      pltpu.make_async_copy(x_hbm.at[idx[p]], rows[p], sems[p]).wait()
      pltpu.sync_copy(rows[p], xe_hbm.at[pl.ds(off, CH)])

    for p in range(2):
      fire(p, p)

    def pair(t, carry):
      c = 2 * t
      for p in range(2):
        drain(p, c + p)
        fire(p, c + 2 + p)
      return carry

    lax.fori_loop(0, (nfull - 3) // 2, pair, 0)
    drain(0, nfull - 3)
    fire(0, nfull - 1)
    drain(1, nfull - 2)

    @pl.when(wid < remw)
    def _tail_fire():
      off = rbase + wid * remr
      pltpu.sync_copy(src_hbm.at[pl.ds(off, remr)], idx_t)
      pltpu.async_copy(x_hbm.at[idx_t], rows_t, sem_t)

    drain(0, nfull - 1)

    @pl.when(wid < remw)
    def _tail_drain():
      off = rbase + wid * remr
      pltpu.make_async_copy(x_hbm.at[idx_t], rows_t, sem_t).wait()
      pltpu.sync_copy(rows_t, xe_hbm.at[pl.ds(off, remr)])

  return functools.partial(
      pl.kernel,
      out_type=jax.ShapeDtypeStruct((ec, D), jnp.float32),
      mesh=_sc_mesh(),
      scratch_types=[
          pltpu.VMEM((CH,), jnp.int32),
          pltpu.VMEM((CH,), jnp.int32),
          pltpu.VMEM((CH, D), jnp.float32),
          pltpu.VMEM((CH, D), jnp.float32),
          pltpu.VMEM((remr,), jnp.int32),
          pltpu.VMEM((remr, D), jnp.float32),
          pltpu.SemaphoreType.DMA,
          pltpu.SemaphoreType.DMA,
          pltpu.SemaphoreType.DMA,
      ],
  )(body)


@functools.cache
def _build_scatter(ec, nchunk):
  epw = ec // NW
  nfull = epw // CH
  tail = epw - nfull * CH
  assert epw % 8 == 0 and tail % 8 == 0 and 0 < tail <= CH and nfull % 2 == 1

  def body(*refs):
    v_hbms = refs[:nchunk]
    (dst_hbm, zeros_hbm, out_hbm, idx0, idx1, rows0, rows1, idx_t, rows_t,
     acc_sh, sem0, sem1, sem_t) = refs[nchunk:]
    cid = lax.axis_index("c")
    sid = lax.axis_index("s")
    pltpu.sync_copy(zeros_hbm.at[pl.ds(sid * RZ, RZ)],
                    acc_sh.at[pl.ds(sid * RZ, RZ)])
    plsc.subcore_barrier()

    wid = sid * NC + cid
    idx = (idx0, idx1)
    rows = (rows0, rows1)
    sems = (sem0, sem1)

    for k in range(nchunk):
      v_hbm = v_hbms[k]
      base = wid * epw
      gbase = k * ec + base

      def fire(p, c):
        pltpu.sync_copy(dst_hbm.at[pl.ds(gbase + c * CH, CH)], idx[p])
        pltpu.async_copy(v_hbm.at[pl.ds(base + c * CH, CH)], rows[p], sems[p])

      def drain(p, c):
        pltpu.make_async_copy(v_hbm.at[pl.ds(base + c * CH, CH)], rows[p],
                              sems[p]).wait()
        pltpu.sync_copy(rows[p], acc_sh.at[idx[p]], add=True)

      for p in range(2):
        fire(p, p)

      def pair(t, carry):
        c = 2 * t
        for p in range(2):
          drain(p, c + p)
          fire(p, c + 2 + p)
        return carry

      lax.fori_loop(0, (nfull - 3) // 2, pair, 0)
      drain(0, nfull - 3)
      fire(0, nfull - 1)
      drain(1, nfull - 2)
      pltpu.sync_copy(dst_hbm.at[pl.ds(gbase + nfull * CH, tail)], idx_t)
      pltpu.async_copy(v_hbm.at[pl.ds(base + nfull * CH, tail)], rows_t, sem_t)
      drain(0, nfull - 1)
      pltpu.make_async_copy(v_hbm.at[pl.ds(base + nfull * CH, tail)], rows_t,
                            sem_t).wait()
      pltpu.sync_copy(rows_t, acc_sh.at[idx_t], add=True)

    plsc.subcore_barrier()
    pltpu.sync_copy(acc_sh.at[pl.ds(sid * RZ, RZ)],
                    out_hbm.at[cid, pl.ds(sid * RZ, RZ)])

  return functools.partial(
      pl.kernel,
      out_type=jax.ShapeDtypeStruct((NC, NP, D), jnp.float32),
      mesh=_sc_mesh(),
      scratch_types=[
          pltpu.VMEM((CH,), jnp.int32),
          pltpu.VMEM((CH,), jnp.int32),
          pltpu.VMEM((CH, D), jnp.float32),
          pltpu.VMEM((CH, D), jnp.float32),
          pltpu.VMEM((tail,), jnp.int32),
          pltpu.VMEM((tail, D), jnp.float32),
          pltpu.VMEM_SHARED((NP, D), jnp.float32),
          pltpu.SemaphoreType.DMA,
          pltpu.SemaphoreType.DMA,
          pltpu.SemaphoreType.DMA,
      ],
  )(body)


def kernel(node_feats, node_attrs_total, node_attrs_slice, edge_feats,
           edge_attrs, edge_index, cutoff, W_up, Wr0, Wr1, Wr2, W_lin, W_sc):
  del node_attrs_slice
  src = edge_index[0]
  dst = edge_index[1]
  Wr2p = Wr2.reshape(H, D, S).transpose(0, 2, 1).reshape(H, S * D)
  W_linp = W_lin.reshape(D, S, D).transpose(1, 0, 2)

  x = _linear_up(node_feats, W_up)
  gather = _build_gather(EC)
  vs = []
  for k in range(NCHUNK):
    lo, hi = k * EC, (k + 1) * EC
    xe = gather(x, src[lo:hi])
    vs.append(_edge_stage(edge_feats[lo:hi], edge_attrs[lo:hi], cutoff[lo:hi],
                          xe, Wr0, Wr1, Wr2p, W_linp))
  zeros = jnp.zeros((NP, D), jnp.float32)
  partial = _build_scatter(EC, NCHUNK)(*vs, dst, zeros)
  m_i, scs = _final_stage(partial, node_attrs_total, W_sc)
  return (m_i, scs)
